# baseline (speedup 1.0000x reference)
import functools

import jax
import jax.numpy as jnp
from jax import lax
from jax.experimental import pallas as pl
from jax.experimental.pallas import tpu as pltpu
from jax.experimental.pallas import tpu_sc as plsc

F32 = jnp.float32

NC = 2
NS = 16
NW = NC * NS
K = 128
G = 2

N = 10000
D = 128
NCLS = 40
TPAD = N + 8
NPART = 10240
ACCR = NPART
RPT = NPART // NS

_mesh = plsc.VectorSubcoreMesh(core_axis_name="c", subcore_axis_name="s")


BCH = 40


def _agg_body(nch, table, src2, dst2, zrows, out, sidx, didx, rows, acc,
              gsA, gsB, ssA, ssB):
  c = lax.axis_index("c")
  s = lax.axis_index("s")
  wid = s * NC + c

  pltpu.sync_copy(zrows, acc.at[pl.ds(s * RPT, RPT)])
  plsc.subcore_barrier()

  gsem = (gsA, gsB)
  ssem = (ssA, ssB)

  def fire_gather(ch, bs):
    pltpu.async_copy(table.at[sidx.at[ch]], rows.at[bs], gsem[bs])

  def wait_gather(ch, bs):
    pltpu.make_async_copy(table.at[sidx.at[ch]], rows.at[bs],
                          gsem[bs]).wait()

  def fire_scatter(ch, bs):
    pltpu.async_copy(rows.at[bs], acc.at[didx.at[ch]], ssem[bs], add=True)

  def wait_scatter(ch, bs):
    pltpu.make_async_copy(rows.at[bs], acc.at[didx.at[ch]], ssem[bs]).wait()

  nsup = BCH // 2
  for blk in range(nch // BCH):
    base = wid * nch + blk * BCH
    pltpu.sync_copy(src2.at[pl.ds(base, BCH)], sidx)
    pltpu.sync_copy(dst2.at[pl.ds(base, BCH)], didx)
    fire_gather(0, 0)

    def body(u, carry):
      a = 2 * u
      b = 2 * u + 1
      wait_gather(a, 0)
      fire_scatter(a, 0)

      @pl.when(u > 0)
      def _():
        wait_scatter(a - 1, 1)

      fire_gather(b, 1)
      wait_gather(b, 1)
      fire_scatter(b, 1)
      wait_scatter(a, 0)

      @pl.when(u < nsup - 1)
      def _():
        fire_gather(a + 2, 0)

      return carry

    lax.fori_loop(0, nsup, body, 0)
    wait_scatter(BCH - 1, 1)

  plsc.subcore_barrier()
  pltpu.sync_copy(acc.at[pl.ds(s * RPT, RPT)], out.at[c, pl.ds(s * RPT, RPT)])


def _make_agg(nch):
  return functools.partial(
      pl.kernel,
      out_type=jax.ShapeDtypeStruct((NC, NPART, D), F32),
      mesh=_mesh,
      scratch_types=[
          pltpu.VMEM((BCH, K), jnp.int32),
          pltpu.VMEM((BCH, K), jnp.int32),
          pltpu.VMEM((2, K, D), F32),
          pltpu.VMEM_SHARED((ACCR, D), F32),
          pltpu.SemaphoreType.DMA,
          pltpu.SemaphoreType.DMA,
          pltpu.SemaphoreType.DMA,
          pltpu.SemaphoreType.DMA,
      ],
  )(functools.partial(_agg_body, nch))


def _deg_body(nch, dst2, ones, zrows, out, didx, ones_v, acc, ssem):
  c = lax.axis_index("c")
  s = lax.axis_index("s")
  wid = s * NC + c

  pltpu.sync_copy(dst2.at[pl.ds(wid * nch, nch)], didx)
  pltpu.sync_copy(ones, ones_v)
  pltpu.sync_copy(zrows, acc.at[pl.ds(s * RPT, RPT)])
  plsc.subcore_barrier()

  def body(u, carry):
    for j in range(4):
      pltpu.async_copy(ones_v, acc.at[didx.at[u * 4 + j]], ssem, add=True)
    for j in range(4):
      pltpu.make_async_copy(ones_v, acc.at[didx.at[u * 4 + j]], ssem).wait()
    return carry

  lax.fori_loop(0, nch // 4, body, 0)
  plsc.subcore_barrier()

  pltpu.sync_copy(acc.at[pl.ds(s * RPT, RPT)], out.at[c, pl.ds(s * RPT, RPT)])


def _make_deg(nch):
  return functools.partial(
      pl.kernel,
      out_type=jax.ShapeDtypeStruct((NC, NPART, D), F32),
      mesh=_mesh,
      scratch_types=[
          pltpu.VMEM((nch, K), jnp.int32),
          pltpu.VMEM((K, D), F32),
          pltpu.VMEM_SHARED((ACCR, D), F32),
          pltpu.SemaphoreType.DMA,
      ],
  )(functools.partial(_deg_body, nch))


_RB = 1000
_GRID = N // _RB


def _k1_body(cnt0, cnt1, x, w, hs, dis16):
  deg = cnt0[:, 0:1] + cnt1[:, 0:1] + 1.0
  dis = lax.rsqrt(deg)
  hs[:] = jnp.dot(x[:], w[:], preferred_element_type=F32) * dis
  dis16[:] = jnp.broadcast_to(dis, (_RB, 16))


def _k1_call(cnt0, cnt1, x, w):
  return pl.pallas_call(
      _k1_body,
      grid=(_GRID,),
      in_specs=[
          pl.BlockSpec((_RB, D), lambda i: (i, 0)),
          pl.BlockSpec((_RB, D), lambda i: (i, 0)),
          pl.BlockSpec((_RB, D), lambda i: (i, 0)),
          pl.BlockSpec((D, D), lambda i: (0, 0)),
      ],
      out_specs=[
          pl.BlockSpec((_RB, D), lambda i: (i, 0)),
          pl.BlockSpec((_RB, 16), lambda i: (i, 0)),
      ],
      out_shape=[
          jax.ShapeDtypeStruct((N, D), F32),
          jax.ShapeDtypeStruct((N, 16), F32),
      ],
  )(cnt0, cnt1, x, w)


def _k2_body(dis16, acc0, acc1, hsp, b, w, o):
  dis = dis16[:, 0:1]
  u = jnp.maximum(dis * (acc0[:] + acc1[:] + hsp[:]) + b[:], 0.0)
  o[:] = jnp.dot(u, w[:], preferred_element_type=F32) * dis


def _k2_call(dis16, acc0, acc1, hsp, b, w):
  return pl.pallas_call(
      _k2_body,
      grid=(_GRID,),
      in_specs=[
          pl.BlockSpec((_RB, 16), lambda i: (i, 0)),
          pl.BlockSpec((_RB, D), lambda i: (i, 0)),
          pl.BlockSpec((_RB, D), lambda i: (i, 0)),
          pl.BlockSpec((_RB, D), lambda i: (i, 0)),
          pl.BlockSpec((1, D), lambda i: (0, 0)),
          pl.BlockSpec((D, D), lambda i: (0, 0)),
      ],
      out_specs=pl.BlockSpec((_RB, D), lambda i: (i, 0)),
      out_shape=jax.ShapeDtypeStruct((N, D), F32),
  )(dis16, acc0, acc1, hsp, b, w)


def _k3_body(dis16, acc0, acc1, hs3, b, o):
  dis = dis16[:, 0:1]
  t = dis * (acc0[:] + acc1[:] + hs3[:]) + b[:]
  col = lax.broadcasted_iota(jnp.int32, t.shape, 1)
  valid = col < NCLS
  tm = jnp.where(valid, t, -jnp.inf)
  m = jnp.max(tm, axis=1, keepdims=True)
  e = jnp.where(valid, jnp.exp(t - m), 0.0)
  lse = jnp.log(jnp.sum(e, axis=1, keepdims=True))
  o[:] = (t - m - lse)[:, :NCLS]


def _k3_call(dis16, acc0, acc1, hs3, b):
  return pl.pallas_call(
      _k3_body,
      grid=(_GRID,),
      in_specs=[
          pl.BlockSpec((_RB, 16), lambda i: (i, 0)),
          pl.BlockSpec((_RB, D), lambda i: (i, 0)),
          pl.BlockSpec((_RB, D), lambda i: (i, 0)),
          pl.BlockSpec((_RB, D), lambda i: (i, 0)),
          pl.BlockSpec((1, D), lambda i: (0, 0)),
      ],
      out_specs=pl.BlockSpec((_RB, NCLS), lambda i: (i, 0)),
      out_shape=jax.ShapeDtypeStruct((N, NCLS), F32),
  )(dis16, acc0, acc1, hs3, b)


def _pad_table(hs):
  return jnp.concatenate([hs, jnp.zeros((TPAD - N, D), F32)], axis=0)


def kernel(x, edge_index, W1, b1, W2, b2, W3, b3):
  E = edge_index.shape[1]
  nch = -(-E // (NW * K))
  nch = -(-nch // BCH) * BCH
  e_pad = nch * NW * K

  src2 = jnp.concatenate(
      [edge_index[0], jnp.full((e_pad - E,), N, jnp.int32)]).reshape(-1, K)
  dst2 = jnp.concatenate(
      [edge_index[1], jnp.full((e_pad - E,), N, jnp.int32)]).reshape(-1, K)

  zrows = jnp.zeros((RPT, D), F32)
  ones_d = jnp.ones((K, D), F32)

  agg = _make_agg(nch)
  deg = _make_deg(nch)

  cnt = deg(dst2, ones_d, zrows)
  hs1, dis16 = _k1_call(cnt[0], cnt[1], x, W1)
  acc1 = agg(_pad_table(hs1), src2, dst2, zrows)
  hs2 = _k2_call(dis16, acc1[0], acc1[1], hs1, b1.reshape(1, D), W2)
  acc2 = agg(_pad_table(hs2), src2, dst2, zrows)

  W3p = jnp.zeros((D, D), F32).at[:, :NCLS].set(W3)
  b3p = jnp.zeros((1, D), F32).at[0, :NCLS].set(b3)
  hs3 = _k2_call(dis16, acc2[0], acc2[1], hs2, b2.reshape(1, D), W3p)
  acc3 = agg(_pad_table(hs3), src2, dst2, zrows)

  return _k3_call(dis16, acc3[0], acc3[1], hs3, b3p)


# speedup vs baseline: 2.7484x; 2.7484x over previous
import functools

import jax
import jax.numpy as jnp
from jax import lax
from jax.experimental import pallas as pl
from jax.experimental.pallas import tpu as pltpu
from jax.experimental.pallas import tpu_sc as plsc

F32 = jnp.float32

NC = 2
NS = 16
NW = NC * NS
K = 128
G = 2

N = 10000
D = 128
NCLS = 40
TPAD = N + 128
NPART = 10240
ACCR = NPART
RPT = NPART // NS

_mesh = plsc.VectorSubcoreMesh(core_axis_name="c", subcore_axis_name="s")


BCH = 40


def _agg_body(nch, table, src2, dst2, zrows, out, sidx, didx, rows, acc,
              gsA, gsB, ssA, ssB):
  c = lax.axis_index("c")
  s = lax.axis_index("s")
  wid = s * NC + c

  pltpu.sync_copy(zrows, acc.at[pl.ds(s * RPT, RPT)])
  plsc.subcore_barrier()

  gsem = (gsA, gsB)
  ssem = (ssA, ssB)

  def fire_gather(ch, bs):
    pltpu.async_copy(table.at[sidx.at[ch]], rows.at[bs], gsem[bs])

  def wait_gather(ch, bs):
    pltpu.make_async_copy(table.at[sidx.at[ch]], rows.at[bs],
                          gsem[bs]).wait()

  def fire_scatter(ch, bs):
    pltpu.async_copy(rows.at[bs], acc.at[didx.at[ch]], ssem[bs], add=True)

  def wait_scatter(ch, bs):
    pltpu.make_async_copy(rows.at[bs], acc.at[didx.at[ch]], ssem[bs]).wait()

  nsup = BCH // 2
  for blk in range(nch // BCH):
    base = wid * nch + blk * BCH
    pltpu.sync_copy(src2.at[pl.ds(base, BCH)], sidx)
    pltpu.sync_copy(dst2.at[pl.ds(base, BCH)], didx)
    fire_gather(0, 0)

    def body(u, carry):
      a = 2 * u
      b = 2 * u + 1
      wait_gather(a, 0)
      fire_scatter(a, 0)

      @pl.when(u > 0)
      def _():
        wait_scatter(a - 1, 1)

      fire_gather(b, 1)
      wait_gather(b, 1)
      fire_scatter(b, 1)
      wait_scatter(a, 0)

      @pl.when(u < nsup - 1)
      def _():
        fire_gather(a + 2, 0)

      return carry

    lax.fori_loop(0, nsup, body, 0)
    wait_scatter(BCH - 1, 1)

  plsc.subcore_barrier()
  pltpu.sync_copy(acc.at[pl.ds(s * RPT, RPT)], out.at[c, pl.ds(s * RPT, RPT)])


def _make_agg(nch):
  return functools.partial(
      pl.kernel,
      out_type=jax.ShapeDtypeStruct((NC, NPART, D), F32),
      mesh=_mesh,
      scratch_types=[
          pltpu.VMEM((BCH, K), jnp.int32),
          pltpu.VMEM((BCH, K), jnp.int32),
          pltpu.VMEM((2, K, D), F32),
          pltpu.VMEM_SHARED((ACCR, D), F32),
          pltpu.SemaphoreType.DMA,
          pltpu.SemaphoreType.DMA,
          pltpu.SemaphoreType.DMA,
          pltpu.SemaphoreType.DMA,
      ],
  )(functools.partial(_agg_body, nch))


def _deg_body(nch, dst2, ones, zrows, out, didx, ones_v, acc, ssem):
  c = lax.axis_index("c")
  s = lax.axis_index("s")
  wid = s * NC + c

  pltpu.sync_copy(dst2.at[pl.ds(wid * nch, nch)], didx)
  pltpu.sync_copy(ones, ones_v)
  pltpu.sync_copy(zrows, acc.at[pl.ds(s * RPT, RPT)])
  plsc.subcore_barrier()

  def body(u, carry):
    for j in range(4):
      pltpu.async_copy(ones_v, acc.at[didx.at[u * 4 + j]], ssem, add=True)
    for j in range(4):
      pltpu.make_async_copy(ones_v, acc.at[didx.at[u * 4 + j]], ssem).wait()
    return carry

  lax.fori_loop(0, nch // 4, body, 0)
  plsc.subcore_barrier()

  pltpu.sync_copy(acc.at[pl.ds(s * RPT, RPT)], out.at[c, pl.ds(s * RPT, RPT)])


def _make_deg(nch):
  return functools.partial(
      pl.kernel,
      out_type=jax.ShapeDtypeStruct((NC, NPART, D), F32),
      mesh=_mesh,
      scratch_types=[
          pltpu.VMEM((nch, K), jnp.int32),
          pltpu.VMEM((K, D), F32),
          pltpu.VMEM_SHARED((ACCR, D), F32),
          pltpu.SemaphoreType.DMA,
      ],
  )(functools.partial(_deg_body, nch))


_RB = 1000
_GRID = N // _RB


def _k1_body(cnt0, cnt1, x, w, hs, dis16):
  deg = cnt0[:, 0:1] + cnt1[:, 0:1] + 1.0
  dis = lax.rsqrt(deg)
  hs[:] = jnp.dot(x[:], w[:], preferred_element_type=F32) * dis
  dis16[:] = jnp.broadcast_to(dis, (_RB, 16))


def _k1_call(cnt0, cnt1, x, w):
  return pl.pallas_call(
      _k1_body,
      grid=(_GRID,),
      in_specs=[
          pl.BlockSpec((_RB, D), lambda i: (i, 0)),
          pl.BlockSpec((_RB, D), lambda i: (i, 0)),
          pl.BlockSpec((_RB, D), lambda i: (i, 0)),
          pl.BlockSpec((D, D), lambda i: (0, 0)),
      ],
      out_specs=[
          pl.BlockSpec((_RB, D), lambda i: (i, 0)),
          pl.BlockSpec((_RB, 16), lambda i: (i, 0)),
      ],
      out_shape=[
          jax.ShapeDtypeStruct((N, D), F32),
          jax.ShapeDtypeStruct((N, 16), F32),
      ],
  )(cnt0, cnt1, x, w)


def _k2_body(dis16, acc0, acc1, hsp, b, w, o):
  dis = dis16[:, 0:1]
  u = jnp.maximum(dis * (acc0[:] + acc1[:] + hsp[:]) + b[:], 0.0)
  o[:] = jnp.dot(u, w[:], preferred_element_type=F32) * dis


def _k2_call(dis16, acc0, acc1, hsp, b, w):
  return pl.pallas_call(
      _k2_body,
      grid=(_GRID,),
      in_specs=[
          pl.BlockSpec((_RB, 16), lambda i: (i, 0)),
          pl.BlockSpec((_RB, D), lambda i: (i, 0)),
          pl.BlockSpec((_RB, D), lambda i: (i, 0)),
          pl.BlockSpec((_RB, D), lambda i: (i, 0)),
          pl.BlockSpec((1, D), lambda i: (0, 0)),
          pl.BlockSpec((D, D), lambda i: (0, 0)),
      ],
      out_specs=pl.BlockSpec((_RB, D), lambda i: (i, 0)),
      out_shape=jax.ShapeDtypeStruct((N, D), F32),
  )(dis16, acc0, acc1, hsp, b, w)


def _k3_body(dis16, acc0, acc1, hs3, b, o):
  dis = dis16[:, 0:1]
  t = dis * (acc0[:] + acc1[:] + hs3[:]) + b[:]
  col = lax.broadcasted_iota(jnp.int32, t.shape, 1)
  valid = col < NCLS
  tm = jnp.where(valid, t, -jnp.inf)
  m = jnp.max(tm, axis=1, keepdims=True)
  e = jnp.where(valid, jnp.exp(t - m), 0.0)
  lse = jnp.log(jnp.sum(e, axis=1, keepdims=True))
  o[:] = (t - m - lse)[:, :NCLS]


def _k3_call(dis16, acc0, acc1, hs3, b):
  return pl.pallas_call(
      _k3_body,
      grid=(_GRID,),
      in_specs=[
          pl.BlockSpec((_RB, 16), lambda i: (i, 0)),
          pl.BlockSpec((_RB, D), lambda i: (i, 0)),
          pl.BlockSpec((_RB, D), lambda i: (i, 0)),
          pl.BlockSpec((_RB, D), lambda i: (i, 0)),
          pl.BlockSpec((1, D), lambda i: (0, 0)),
      ],
      out_specs=pl.BlockSpec((_RB, NCLS), lambda i: (i, 0)),
      out_shape=jax.ShapeDtypeStruct((N, NCLS), F32),
  )(dis16, acc0, acc1, hs3, b)


def _pad_table(hs):
  return jnp.concatenate([hs, jnp.zeros((TPAD - N, D), F32)], axis=0)


def kernel(x, edge_index, W1, b1, W2, b2, W3, b3):
  E = edge_index.shape[1]
  nch = -(-E // (NW * K))
  nch = -(-nch // BCH) * BCH
  e_pad = nch * NW * K

  pad_ar = jnp.arange(e_pad - E, dtype=jnp.int32)
  src2 = jnp.concatenate(
      [edge_index[0], N + pad_ar % (TPAD - N)]).reshape(-1, K)
  dst2 = jnp.concatenate(
      [edge_index[1], N + pad_ar % (NPART - N)]).reshape(-1, K)

  zrows = jnp.zeros((RPT, D), F32)
  ones_d = jnp.ones((K, D), F32)

  agg = _make_agg(nch)
  deg = _make_deg(nch)

  cnt = deg(dst2, ones_d, zrows)
  hs1, dis16 = _k1_call(cnt[0], cnt[1], x, W1)
  acc1 = agg(_pad_table(hs1), src2, dst2, zrows)
  hs2 = _k2_call(dis16, acc1[0], acc1[1], hs1, b1.reshape(1, D), W2)
  acc2 = agg(_pad_table(hs2), src2, dst2, zrows)

  W3p = jnp.zeros((D, D), F32).at[:, :NCLS].set(W3)
  b3p = jnp.zeros((1, D), F32).at[0, :NCLS].set(b3)
  hs3 = _k2_call(dis16, acc2[0], acc2[1], hs2, b2.reshape(1, D), W3p)
  acc3 = agg(_pad_table(hs3), src2, dst2, zrows)

  return _k3_call(dis16, acc3[0], acc3[1], hs3, b3p)


# speedup vs baseline: 2.9589x; 1.0766x over previous
import functools

import jax
import jax.numpy as jnp
from jax import lax
from jax.experimental import pallas as pl
from jax.experimental.pallas import tpu as pltpu
from jax.experimental.pallas import tpu_sc as plsc

F32 = jnp.float32

NC = 2
NS = 16
NW = NC * NS
K = 128
G = 2

N = 10000
D = 128
NCLS = 40
NPART = 10240
TPAD = NPART
ACCR = NPART
RPT = NPART // NS

_mesh = plsc.VectorSubcoreMesh(core_axis_name="c", subcore_axis_name="s")


BCH = 40


def _agg_body(nch, table, src2, dst2, zrows, out, sidx, didx, rows, acc,
              gsA, gsB, ssA, ssB):
  c = lax.axis_index("c")
  s = lax.axis_index("s")
  wid = s * NC + c

  pltpu.sync_copy(zrows, acc.at[pl.ds(s * RPT, RPT)])
  plsc.subcore_barrier()

  gsem = (gsA, gsB)
  ssem = (ssA, ssB)

  def fire_gather(ch, bs):
    pltpu.async_copy(table.at[sidx.at[ch]], rows.at[bs], gsem[bs])

  def wait_gather(ch, bs):
    pltpu.make_async_copy(table.at[sidx.at[ch]], rows.at[bs],
                          gsem[bs]).wait()

  def fire_scatter(ch, bs):
    pltpu.async_copy(rows.at[bs], acc.at[didx.at[ch]], ssem[bs], add=True)

  def wait_scatter(ch, bs):
    pltpu.make_async_copy(rows.at[bs], acc.at[didx.at[ch]], ssem[bs]).wait()

  nsup = BCH // 2
  for blk in range(nch // BCH):
    base = wid * nch + blk * BCH
    pltpu.sync_copy(src2.at[pl.ds(base, BCH)], sidx)
    pltpu.sync_copy(dst2.at[pl.ds(base, BCH)], didx)
    fire_gather(0, 0)

    def body(u, carry):
      a = 2 * u
      b = 2 * u + 1
      wait_gather(a, 0)
      fire_scatter(a, 0)

      @pl.when(u > 0)
      def _():
        wait_scatter(a - 1, 1)

      fire_gather(b, 1)
      wait_gather(b, 1)
      fire_scatter(b, 1)
      wait_scatter(a, 0)

      @pl.when(u < nsup - 1)
      def _():
        fire_gather(a + 2, 0)

      return carry

    lax.fori_loop(0, nsup, body, 0)
    wait_scatter(BCH - 1, 1)

  plsc.subcore_barrier()
  pltpu.sync_copy(acc.at[pl.ds(s * RPT, RPT)], out.at[c, pl.ds(s * RPT, RPT)])


def _make_agg(nch):
  return functools.partial(
      pl.kernel,
      out_type=jax.ShapeDtypeStruct((NC, NPART, D), F32),
      mesh=_mesh,
      scratch_types=[
          pltpu.VMEM((BCH, K), jnp.int32),
          pltpu.VMEM((BCH, K), jnp.int32),
          pltpu.VMEM((2, K, D), F32),
          pltpu.VMEM_SHARED((ACCR, D), F32),
          pltpu.SemaphoreType.DMA,
          pltpu.SemaphoreType.DMA,
          pltpu.SemaphoreType.DMA,
          pltpu.SemaphoreType.DMA,
      ],
  )(functools.partial(_agg_body, nch))


def _deg_body(nch, dst2, ones, zrows, out, didx, ones_v, acc, ssem):
  c = lax.axis_index("c")
  s = lax.axis_index("s")
  wid = s * NC + c

  pltpu.sync_copy(dst2.at[pl.ds(wid * nch, nch)], didx)
  pltpu.sync_copy(ones, ones_v)
  pltpu.sync_copy(zrows, acc.at[pl.ds(s * RPT, RPT)])
  plsc.subcore_barrier()

  def body(u, carry):
    for j in range(4):
      pltpu.async_copy(ones_v, acc.at[didx.at[u * 4 + j]], ssem, add=True)
    for j in range(4):
      pltpu.make_async_copy(ones_v, acc.at[didx.at[u * 4 + j]], ssem).wait()
    return carry

  lax.fori_loop(0, nch // 4, body, 0)
  plsc.subcore_barrier()

  pltpu.sync_copy(acc.at[pl.ds(s * RPT, RPT)], out.at[c, pl.ds(s * RPT, RPT)])


def _make_deg(nch):
  return functools.partial(
      pl.kernel,
      out_type=jax.ShapeDtypeStruct((NC, NPART, D), F32),
      mesh=_mesh,
      scratch_types=[
          pltpu.VMEM((nch, K), jnp.int32),
          pltpu.VMEM((K, D), F32),
          pltpu.VMEM_SHARED((ACCR, D), F32),
          pltpu.SemaphoreType.DMA,
      ],
  )(functools.partial(_deg_body, nch))


_RB = 1280
_GRID = NPART // _RB


def _k1_body(cnt, x, w, hs, dis16):
  deg = cnt[0, :, 0:1] + cnt[1, :, 0:1] + 1.0
  dis = lax.rsqrt(deg)
  hs[:] = jnp.dot(x[:], w[:], preferred_element_type=F32) * dis
  dis16[:] = jnp.broadcast_to(dis, (_RB, 16))


def _k1_call(cnt, xp, w):
  return pl.pallas_call(
      _k1_body,
      grid=(_GRID,),
      in_specs=[
          pl.BlockSpec((2, _RB, D), lambda i: (0, i, 0)),
          pl.BlockSpec((_RB, D), lambda i: (i, 0)),
          pl.BlockSpec((D, D), lambda i: (0, 0)),
      ],
      out_specs=[
          pl.BlockSpec((_RB, D), lambda i: (i, 0)),
          pl.BlockSpec((_RB, 16), lambda i: (i, 0)),
      ],
      out_shape=[
          jax.ShapeDtypeStruct((NPART, D), F32),
          jax.ShapeDtypeStruct((NPART, 16), F32),
      ],
  )(cnt, xp, w)


def _k2_body(dis16, acc, hsp, b, w, o):
  i = pl.program_id(0)
  dis = dis16[:, 0:1]
  u = jnp.maximum(dis * (acc[0] + acc[1] + hsp[:]) + b[:], 0.0)
  row = i * _RB + lax.broadcasted_iota(jnp.int32, (_RB, D), 0)
  u = jnp.where(row < N, u, 0.0)
  o[:] = jnp.dot(u, w[:], preferred_element_type=F32) * dis


def _k2_call(dis16, acc, hsp, b, w):
  return pl.pallas_call(
      _k2_body,
      grid=(_GRID,),
      in_specs=[
          pl.BlockSpec((_RB, 16), lambda i: (i, 0)),
          pl.BlockSpec((2, _RB, D), lambda i: (0, i, 0)),
          pl.BlockSpec((_RB, D), lambda i: (i, 0)),
          pl.BlockSpec((1, D), lambda i: (0, 0)),
          pl.BlockSpec((D, D), lambda i: (0, 0)),
      ],
      out_specs=pl.BlockSpec((_RB, D), lambda i: (i, 0)),
      out_shape=jax.ShapeDtypeStruct((NPART, D), F32),
  )(dis16, acc, hsp, b, w)


_RB3 = 1000
_GRID3 = N // _RB3


def _k3_body(dis16, acc, hs3, b, o):
  dis = dis16[:, 0:1]
  t = dis * (acc[0] + acc[1] + hs3[:]) + b[:]
  col = lax.broadcasted_iota(jnp.int32, t.shape, 1)
  valid = col < NCLS
  tm = jnp.where(valid, t, -jnp.inf)
  m = jnp.max(tm, axis=1, keepdims=True)
  e = jnp.where(valid, jnp.exp(t - m), 0.0)
  lse = jnp.log(jnp.sum(e, axis=1, keepdims=True))
  o[:] = (t - m - lse)[:, :NCLS]


def _k3_call(dis16, acc, hs3, b):
  return pl.pallas_call(
      _k3_body,
      grid=(_GRID3,),
      in_specs=[
          pl.BlockSpec((_RB3, 16), lambda i: (i, 0)),
          pl.BlockSpec((2, _RB3, D), lambda i: (0, i, 0)),
          pl.BlockSpec((_RB3, D), lambda i: (i, 0)),
          pl.BlockSpec((1, D), lambda i: (0, 0)),
      ],
      out_specs=pl.BlockSpec((_RB3, NCLS), lambda i: (i, 0)),
      out_shape=jax.ShapeDtypeStruct((N, NCLS), F32),
  )(dis16, acc, hs3, b)


def kernel(x, edge_index, W1, b1, W2, b2, W3, b3):
  E = edge_index.shape[1]
  nch = -(-E // (NW * K))
  nch = -(-nch // BCH) * BCH
  e_pad = nch * NW * K

  pad_ar = jnp.arange(e_pad - E, dtype=jnp.int32)
  src2 = jnp.concatenate(
      [edge_index[0], N + pad_ar % (TPAD - N)]).reshape(-1, K)
  dst2 = jnp.concatenate(
      [edge_index[1], N + pad_ar % (NPART - N)]).reshape(-1, K)

  zrows = jnp.zeros((RPT, D), F32)
  ones_d = jnp.ones((K, D), F32)
  xp = jnp.zeros((NPART, D), F32).at[:N].set(x)

  agg = _make_agg(nch)
  deg = _make_deg(nch)

  cnt = deg(dst2, ones_d, zrows)
  hs1, dis16 = _k1_call(cnt, xp, W1)
  acc1 = agg(hs1, src2, dst2, zrows)
  hs2 = _k2_call(dis16, acc1, hs1, b1.reshape(1, D), W2)
  acc2 = agg(hs2, src2, dst2, zrows)

  W3p = jnp.zeros((D, D), F32).at[:, :NCLS].set(W3)
  b3p = jnp.zeros((1, D), F32).at[0, :NCLS].set(b3)
  hs3 = _k2_call(dis16, acc2, hs2, b2.reshape(1, D), W3p)
  acc3 = agg(hs3, src2, dst2, zrows)

  return _k3_call(dis16, acc3, hs3, b3p)


# speedup vs baseline: 3.3209x; 1.1223x over previous
import functools

import jax
import jax.numpy as jnp
from jax import lax
from jax.experimental import pallas as pl
from jax.experimental.pallas import tpu as pltpu
from jax.experimental.pallas import tpu_sc as plsc

F32 = jnp.float32

NC = 2
NS = 16
NW = NC * NS
K = 128
G = 2

N = 10000
D = 128
NCLS = 40
NPART = 10240
TPAD = NPART
ACCR = NPART
RPT = NPART // NS

_mesh = plsc.VectorSubcoreMesh(core_axis_name="c", subcore_axis_name="s")


BCH = 40


def _agg_body(nch, table, src2, dst2, zrows, out, sidx, didx, rows, acc,
              gsA, gsB, ssA, ssB):
  c = lax.axis_index("c")
  s = lax.axis_index("s")
  wid = s * NC + c

  pltpu.sync_copy(zrows, acc.at[pl.ds(s * RPT, RPT)])
  plsc.subcore_barrier()

  gsem = (gsA, gsB)
  ssem = (ssA, ssB)

  def fire_gather(ch, bs):
    pltpu.async_copy(table.at[sidx.at[ch]], rows.at[bs], gsem[bs])

  def wait_gather(ch, bs):
    pltpu.make_async_copy(table.at[sidx.at[ch]], rows.at[bs],
                          gsem[bs]).wait()

  def fire_scatter(ch, bs):
    pltpu.async_copy(rows.at[bs], acc.at[didx.at[ch]], ssem[bs], add=True)

  def wait_scatter(ch, bs):
    pltpu.make_async_copy(rows.at[bs], acc.at[didx.at[ch]], ssem[bs]).wait()

  nsup = BCH // 2
  for blk in range(nch // BCH):
    base = wid * nch + blk * BCH
    pltpu.sync_copy(src2.at[pl.ds(base, BCH)], sidx)
    pltpu.sync_copy(dst2.at[pl.ds(base, BCH)], didx)
    fire_gather(0, 0)

    def body(u, carry):
      a = 2 * u
      b = 2 * u + 1
      wait_gather(a, 0)
      fire_scatter(a, 0)

      @pl.when(u > 0)
      def _():
        wait_scatter(a - 1, 1)

      fire_gather(b, 1)
      wait_gather(b, 1)
      fire_scatter(b, 1)
      wait_scatter(a, 0)

      @pl.when(u < nsup - 1)
      def _():
        fire_gather(a + 2, 0)

      return carry

    lax.fori_loop(0, nsup, body, 0)
    wait_scatter(BCH - 1, 1)

  plsc.subcore_barrier()
  pltpu.sync_copy(acc.at[pl.ds(s * RPT, RPT)], out.at[c, pl.ds(s * RPT, RPT)])


def _make_agg(nch):
  return functools.partial(
      pl.kernel,
      out_type=jax.ShapeDtypeStruct((NC, NPART, D), F32),
      mesh=_mesh,
      scratch_types=[
          pltpu.VMEM((BCH, K), jnp.int32),
          pltpu.VMEM((BCH, K), jnp.int32),
          pltpu.VMEM((2, K, D), F32),
          pltpu.VMEM_SHARED((ACCR, D), F32),
          pltpu.SemaphoreType.DMA,
          pltpu.SemaphoreType.DMA,
          pltpu.SemaphoreType.DMA,
          pltpu.SemaphoreType.DMA,
      ],
  )(functools.partial(_agg_body, nch))


def _deg_body(nch, dst2, ones, zrows, out, didx, ones_v, acc, ssem):
  c = lax.axis_index("c")
  s = lax.axis_index("s")
  wid = s * NC + c

  pltpu.sync_copy(dst2.at[pl.ds(wid * nch, nch)], didx)
  pltpu.sync_copy(ones, ones_v)
  pltpu.sync_copy(zrows, acc.at[pl.ds(s * RPT, RPT)])
  plsc.subcore_barrier()

  def body(u, carry):
    for j in range(4):
      pltpu.async_copy(ones_v, acc.at[didx.at[u * 4 + j]], ssem, add=True)
    for j in range(4):
      pltpu.make_async_copy(ones_v, acc.at[didx.at[u * 4 + j]], ssem).wait()
    return carry

  lax.fori_loop(0, nch // 4, body, 0)
  plsc.subcore_barrier()

  pltpu.sync_copy(acc.at[pl.ds(s * RPT, RPT)], out.at[c, pl.ds(s * RPT, RPT)])


def _make_deg(nch):
  return functools.partial(
      pl.kernel,
      out_type=jax.ShapeDtypeStruct((NC, NPART), jnp.int32),
      mesh=_mesh,
      scratch_types=[
          pltpu.VMEM((nch, K), jnp.int32),
          pltpu.VMEM((K,), jnp.int32),
          pltpu.VMEM_SHARED((ACCR,), jnp.int32),
          pltpu.SemaphoreType.DMA,
      ],
  )(functools.partial(_deg_body, nch))


_RB = 1280
_GRID = NPART // _RB


def _k1_body(degcol, x, w, hs, dis16):
  dis = lax.rsqrt(degcol[:])
  hs[:] = jnp.dot(x[:], w[:], preferred_element_type=F32) * dis
  dis16[:] = jnp.broadcast_to(dis, (_RB, 16))


def _k1_call(degcol, xp, w):
  return pl.pallas_call(
      _k1_body,
      grid=(_GRID,),
      in_specs=[
          pl.BlockSpec((_RB, 1), lambda i: (i, 0)),
          pl.BlockSpec((_RB, D), lambda i: (i, 0)),
          pl.BlockSpec((D, D), lambda i: (0, 0)),
      ],
      out_specs=[
          pl.BlockSpec((_RB, D), lambda i: (i, 0)),
          pl.BlockSpec((_RB, 16), lambda i: (i, 0)),
      ],
      out_shape=[
          jax.ShapeDtypeStruct((NPART, D), F32),
          jax.ShapeDtypeStruct((NPART, 16), F32),
      ],
  )(degcol, xp, w)


def _k2_body(dis16, acc, hsp, b, w, o):
  i = pl.program_id(0)
  dis = dis16[:, 0:1]
  u = jnp.maximum(dis * (acc[0] + acc[1] + hsp[:]) + b[:], 0.0)
  row = i * _RB + lax.broadcasted_iota(jnp.int32, (_RB, D), 0)
  u = jnp.where(row < N, u, 0.0)
  o[:] = jnp.dot(u, w[:], preferred_element_type=F32) * dis


def _k2_call(dis16, acc, hsp, b, w):
  return pl.pallas_call(
      _k2_body,
      grid=(_GRID,),
      in_specs=[
          pl.BlockSpec((_RB, 16), lambda i: (i, 0)),
          pl.BlockSpec((2, _RB, D), lambda i: (0, i, 0)),
          pl.BlockSpec((_RB, D), lambda i: (i, 0)),
          pl.BlockSpec((1, D), lambda i: (0, 0)),
          pl.BlockSpec((D, D), lambda i: (0, 0)),
      ],
      out_specs=pl.BlockSpec((_RB, D), lambda i: (i, 0)),
      out_shape=jax.ShapeDtypeStruct((NPART, D), F32),
  )(dis16, acc, hsp, b, w)


_RB3 = 1000
_GRID3 = N // _RB3


def _k3_body(dis16, acc, hs3, b, o):
  dis = dis16[:, 0:1]
  t = dis * (acc[0] + acc[1] + hs3[:]) + b[:]
  col = lax.broadcasted_iota(jnp.int32, t.shape, 1)
  valid = col < NCLS
  tm = jnp.where(valid, t, -jnp.inf)
  m = jnp.max(tm, axis=1, keepdims=True)
  e = jnp.where(valid, jnp.exp(t - m), 0.0)
  lse = jnp.log(jnp.sum(e, axis=1, keepdims=True))
  o[:] = (t - m - lse)[:, :NCLS]


def _k3_call(dis16, acc, hs3, b):
  return pl.pallas_call(
      _k3_body,
      grid=(_GRID3,),
      in_specs=[
          pl.BlockSpec((_RB3, 16), lambda i: (i, 0)),
          pl.BlockSpec((2, _RB3, D), lambda i: (0, i, 0)),
          pl.BlockSpec((_RB3, D), lambda i: (i, 0)),
          pl.BlockSpec((1, D), lambda i: (0, 0)),
      ],
      out_specs=pl.BlockSpec((_RB3, NCLS), lambda i: (i, 0)),
      out_shape=jax.ShapeDtypeStruct((N, NCLS), F32),
  )(dis16, acc, hs3, b)


def kernel(x, edge_index, W1, b1, W2, b2, W3, b3):
  E = edge_index.shape[1]
  nch = -(-E // (NW * K))
  nch = -(-nch // BCH) * BCH
  e_pad = nch * NW * K

  pad_ar = jnp.arange(e_pad - E, dtype=jnp.int32)
  src2 = jnp.concatenate(
      [edge_index[0], N + pad_ar % (TPAD - N)]).reshape(-1, K)
  dst2 = jnp.concatenate(
      [edge_index[1], N + pad_ar % (NPART - N)]).reshape(-1, K)

  zrows = jnp.zeros((RPT, D), F32)
  zcnt = jnp.zeros((RPT,), jnp.int32)
  ones1 = jnp.ones((K,), jnp.int32)
  xp = jnp.zeros((NPART, D), F32).at[:N].set(x)

  agg = _make_agg(nch)
  deg = _make_deg(nch)

  cnt = deg(dst2, ones1, zcnt)
  degcol = (cnt[0] + cnt[1] + 1).astype(F32).reshape(NPART, 1)
  hs1, dis16 = _k1_call(degcol, xp, W1)
  acc1 = agg(hs1, src2, dst2, zrows)
  hs2 = _k2_call(dis16, acc1, hs1, b1.reshape(1, D), W2)
  acc2 = agg(hs2, src2, dst2, zrows)

  W3p = jnp.zeros((D, D), F32).at[:, :NCLS].set(W3)
  b3p = jnp.zeros((1, D), F32).at[0, :NCLS].set(b3)
  hs3 = _k2_call(dis16, acc2, hs2, b2.reshape(1, D), W3p)
  acc3 = agg(hs3, src2, dst2, zrows)

  return _k3_call(dis16, acc3, hs3, b3p)


# speedup vs baseline: 3.6595x; 1.1019x over previous
import functools

import jax
import jax.numpy as jnp
from jax import lax
from jax.experimental import pallas as pl
from jax.experimental.pallas import tpu as pltpu
from jax.experimental.pallas import tpu_sc as plsc

F32 = jnp.float32

NC = 2
NS = 16
NW = NC * NS
K = 128
G = 2

N = 10000
D = 128
NCLS = 40
NPART = 10240
TPAD = NPART
ACCR = NPART
RPT = NPART // NS

_mesh = plsc.VectorSubcoreMesh(core_axis_name="c", subcore_axis_name="s")


BCH = 40


KK = 64
NB = 4


def _agg_body(nch, table, src2, dst2, zrows, out, sidx, didx, rows, acc,
              gs0, gs1, gs2, gs3, ss0, ss1, ss2, ss3):
  c = lax.axis_index("c")
  s = lax.axis_index("s")
  wid = s * NC + c

  pltpu.sync_copy(zrows, acc.at[pl.ds(s * RPT, RPT)])
  plsc.subcore_barrier()

  gsem = (gs0, gs1, gs2, gs3)
  ssem = (ss0, ss1, ss2, ss3)

  def refs(u, j):
    r = 2 * u + (j // 2)
    h = (j % 2) * KK
    return sidx.at[r, pl.ds(h, KK)], didx.at[r, pl.ds(h, KK)]

  def fire_gather(u, j):
    si, _ = refs(u, j)
    pltpu.async_copy(table.at[si], rows.at[j], gsem[j])

  def wait_gather(u, j):
    si, _ = refs(u, j)
    pltpu.make_async_copy(table.at[si], rows.at[j], gsem[j]).wait()

  def fire_scatter(u, j):
    _, di = refs(u, j)
    pltpu.async_copy(rows.at[j], acc.at[di], ssem[j], add=True)

  def wait_scatter(u, j):
    _, di = refs(u, j)
    pltpu.make_async_copy(rows.at[j], acc.at[di], ssem[j]).wait()

  nsup = (BCH * 2) // NB
  for blk in range(nch // BCH):
    base = wid * nch + blk * BCH
    pltpu.sync_copy(src2.at[pl.ds(base, BCH)], sidx)
    pltpu.sync_copy(dst2.at[pl.ds(base, BCH)], didx)
    for j in range(NB):
      fire_gather(0, j)

    def body(u, carry):
      for j in range(NB):
        wait_gather(u, j)
        fire_scatter(u, j)
      for j in range(NB):
        wait_scatter(u, j)

        @pl.when(u < nsup - 1)
        def _():
          fire_gather(u + 1, j)

      return carry

    lax.fori_loop(0, nsup, body, 0)

  plsc.subcore_barrier()
  pltpu.sync_copy(acc.at[pl.ds(s * RPT, RPT)], out.at[c, pl.ds(s * RPT, RPT)])


def _make_agg(nch):
  return functools.partial(
      pl.kernel,
      out_type=jax.ShapeDtypeStruct((NC, NPART, D), F32),
      mesh=_mesh,
      scratch_types=[
          pltpu.VMEM((BCH, K), jnp.int32),
          pltpu.VMEM((BCH, K), jnp.int32),
          pltpu.VMEM((NB, KK, D), F32),
          pltpu.VMEM_SHARED((ACCR, D), F32),
          pltpu.SemaphoreType.DMA,
          pltpu.SemaphoreType.DMA,
          pltpu.SemaphoreType.DMA,
          pltpu.SemaphoreType.DMA,
          pltpu.SemaphoreType.DMA,
          pltpu.SemaphoreType.DMA,
          pltpu.SemaphoreType.DMA,
          pltpu.SemaphoreType.DMA,
      ],
  )(functools.partial(_agg_body, nch))


def _deg_body(nch, dst2, ones, zrows, out, didx, ones_v, acc, ssem):
  c = lax.axis_index("c")
  s = lax.axis_index("s")
  wid = s * NC + c

  pltpu.sync_copy(dst2.at[pl.ds(wid * nch, nch)], didx)
  pltpu.sync_copy(ones, ones_v)
  pltpu.sync_copy(zrows, acc.at[pl.ds(s * RPT, RPT)])
  plsc.subcore_barrier()

  def body(u, carry):
    for j in range(4):
      pltpu.async_copy(ones_v, acc.at[didx.at[u * 4 + j]], ssem, add=True)
    for j in range(4):
      pltpu.make_async_copy(ones_v, acc.at[didx.at[u * 4 + j]], ssem).wait()
    return carry

  lax.fori_loop(0, nch // 4, body, 0)
  plsc.subcore_barrier()

  pltpu.sync_copy(acc.at[pl.ds(s * RPT, RPT)], out.at[c, pl.ds(s * RPT, RPT)])


def _make_deg(nch):
  return functools.partial(
      pl.kernel,
      out_type=jax.ShapeDtypeStruct((NC, NPART), jnp.int32),
      mesh=_mesh,
      scratch_types=[
          pltpu.VMEM((nch, K), jnp.int32),
          pltpu.VMEM((K,), jnp.int32),
          pltpu.VMEM_SHARED((ACCR,), jnp.int32),
          pltpu.SemaphoreType.DMA,
      ],
  )(functools.partial(_deg_body, nch))


_RB = 1280
_GRID = NPART // _RB


def _k1_body(degcol, x, w, hs, dis16):
  dis = lax.rsqrt(degcol[:])
  hs[:] = jnp.dot(x[:], w[:], preferred_element_type=F32) * dis
  dis16[:] = jnp.broadcast_to(dis, (_RB, 16))


def _k1_call(degcol, xp, w):
  return pl.pallas_call(
      _k1_body,
      grid=(_GRID,),
      in_specs=[
          pl.BlockSpec((_RB, 1), lambda i: (i, 0)),
          pl.BlockSpec((_RB, D), lambda i: (i, 0)),
          pl.BlockSpec((D, D), lambda i: (0, 0)),
      ],
      out_specs=[
          pl.BlockSpec((_RB, D), lambda i: (i, 0)),
          pl.BlockSpec((_RB, 16), lambda i: (i, 0)),
      ],
      out_shape=[
          jax.ShapeDtypeStruct((NPART, D), F32),
          jax.ShapeDtypeStruct((NPART, 16), F32),
      ],
  )(degcol, xp, w)


def _k2_body(dis16, acc, hsp, b, w, o):
  i = pl.program_id(0)
  dis = dis16[:, 0:1]
  u = jnp.maximum(dis * (acc[0] + acc[1] + hsp[:]) + b[:], 0.0)
  row = i * _RB + lax.broadcasted_iota(jnp.int32, (_RB, D), 0)
  u = jnp.where(row < N, u, 0.0)
  o[:] = jnp.dot(u, w[:], preferred_element_type=F32) * dis


def _k2_call(dis16, acc, hsp, b, w):
  return pl.pallas_call(
      _k2_body,
      grid=(_GRID,),
      in_specs=[
          pl.BlockSpec((_RB, 16), lambda i: (i, 0)),
          pl.BlockSpec((2, _RB, D), lambda i: (0, i, 0)),
          pl.BlockSpec((_RB, D), lambda i: (i, 0)),
          pl.BlockSpec((1, D), lambda i: (0, 0)),
          pl.BlockSpec((D, D), lambda i: (0, 0)),
      ],
      out_specs=pl.BlockSpec((_RB, D), lambda i: (i, 0)),
      out_shape=jax.ShapeDtypeStruct((NPART, D), F32),
  )(dis16, acc, hsp, b, w)


_RB3 = 1000
_GRID3 = N // _RB3


def _k3_body(dis16, acc, hs3, b, o):
  dis = dis16[:, 0:1]
  t = dis * (acc[0] + acc[1] + hs3[:]) + b[:]
  col = lax.broadcasted_iota(jnp.int32, t.shape, 1)
  valid = col < NCLS
  tm = jnp.where(valid, t, -jnp.inf)
  m = jnp.max(tm, axis=1, keepdims=True)
  e = jnp.where(valid, jnp.exp(t - m), 0.0)
  lse = jnp.log(jnp.sum(e, axis=1, keepdims=True))
  o[:] = (t - m - lse)[:, :NCLS]


def _k3_call(dis16, acc, hs3, b):
  return pl.pallas_call(
      _k3_body,
      grid=(_GRID3,),
      in_specs=[
          pl.BlockSpec((_RB3, 16), lambda i: (i, 0)),
          pl.BlockSpec((2, _RB3, D), lambda i: (0, i, 0)),
          pl.BlockSpec((_RB3, D), lambda i: (i, 0)),
          pl.BlockSpec((1, D), lambda i: (0, 0)),
      ],
      out_specs=pl.BlockSpec((_RB3, NCLS), lambda i: (i, 0)),
      out_shape=jax.ShapeDtypeStruct((N, NCLS), F32),
  )(dis16, acc, hs3, b)


def kernel(x, edge_index, W1, b1, W2, b2, W3, b3):
  E = edge_index.shape[1]
  nch = -(-E // (NW * K))
  nch = -(-nch // BCH) * BCH
  e_pad = nch * NW * K

  pad_ar = jnp.arange(e_pad - E, dtype=jnp.int32)
  src2 = jnp.concatenate(
      [edge_index[0], N + pad_ar % (TPAD - N)]).reshape(-1, K)
  dst2 = jnp.concatenate(
      [edge_index[1], N + pad_ar % (NPART - N)]).reshape(-1, K)

  zrows = jnp.zeros((RPT, D), F32)
  zcnt = jnp.zeros((RPT,), jnp.int32)
  ones1 = jnp.ones((K,), jnp.int32)
  xp = jnp.zeros((NPART, D), F32).at[:N].set(x)

  agg = _make_agg(nch)
  deg = _make_deg(nch)

  cnt = deg(dst2, ones1, zcnt)
  degcol = (cnt[0] + cnt[1] + 1).astype(F32).reshape(NPART, 1)
  hs1, dis16 = _k1_call(degcol, xp, W1)
  acc1 = agg(hs1, src2, dst2, zrows)
  hs2 = _k2_call(dis16, acc1, hs1, b1.reshape(1, D), W2)
  acc2 = agg(hs2, src2, dst2, zrows)

  W3p = jnp.zeros((D, D), F32).at[:, :NCLS].set(W3)
  b3p = jnp.zeros((1, D), F32).at[0, :NCLS].set(b3)
  hs3 = _k2_call(dis16, acc2, hs2, b2.reshape(1, D), W3p)
  acc3 = agg(hs3, src2, dst2, zrows)

  return _k3_call(dis16, acc3, hs3, b3p)


# speedup vs baseline: 3.7299x; 1.0192x over previous
import functools

import jax
import jax.numpy as jnp
from jax import lax
from jax.experimental import pallas as pl
from jax.experimental.pallas import tpu as pltpu
from jax.experimental.pallas import tpu_sc as plsc

F32 = jnp.float32

NC = 2
NS = 16
NW = NC * NS
K = 128
G = 2

N = 10000
D = 128
NCLS = 40
NPART = 10240
TPAD = NPART
ACCR = NPART
RPT = NPART // NS

_mesh = plsc.VectorSubcoreMesh(core_axis_name="c", subcore_axis_name="s")


BCH = 40


KK = 32
NB = 8
_SPR = K // KK


def _agg_body(nch, table, src2, dst2, zrows, out, sidx, didx, rows, acc,
              gs0, gs1, gs2, gs3, gs4, gs5, gs6, gs7,
              ss0, ss1, ss2, ss3, ss4, ss5, ss6, ss7):
  c = lax.axis_index("c")
  s = lax.axis_index("s")
  wid = s * NC + c

  pltpu.sync_copy(zrows, acc.at[pl.ds(s * RPT, RPT)])
  plsc.subcore_barrier()

  gsem = (gs0, gs1, gs2, gs3, gs4, gs5, gs6, gs7)
  ssem = (ss0, ss1, ss2, ss3, ss4, ss5, ss6, ss7)

  def refs(u, j):
    r = (NB // _SPR) * u + (j // _SPR)
    h = (j % _SPR) * KK
    return sidx.at[r, pl.ds(h, KK)], didx.at[r, pl.ds(h, KK)]

  def fire_gather(u, j):
    si, _ = refs(u, j)
    pltpu.async_copy(table.at[si], rows.at[j], gsem[j])

  def wait_gather(u, j):
    si, _ = refs(u, j)
    pltpu.make_async_copy(table.at[si], rows.at[j], gsem[j]).wait()

  def fire_scatter(u, j):
    _, di = refs(u, j)
    pltpu.async_copy(rows.at[j], acc.at[di], ssem[j], add=True)

  def wait_scatter(u, j):
    _, di = refs(u, j)
    pltpu.make_async_copy(rows.at[j], acc.at[di], ssem[j]).wait()

  nsup = (BCH * _SPR) // NB
  for blk in range(nch // BCH):
    base = wid * nch + blk * BCH
    pltpu.sync_copy(src2.at[pl.ds(base, BCH)], sidx)
    pltpu.sync_copy(dst2.at[pl.ds(base, BCH)], didx)
    for j in range(NB):
      fire_gather(0, j)

    def body(u, carry):
      for j in range(NB):
        wait_gather(u, j)
        fire_scatter(u, j)
      for j in range(NB):
        wait_scatter(u, j)

        @pl.when(u < nsup - 1)
        def _():
          fire_gather(u + 1, j)

      return carry

    lax.fori_loop(0, nsup, body, 0)

  plsc.subcore_barrier()
  pltpu.sync_copy(acc.at[pl.ds(s * RPT, RPT)], out.at[c, pl.ds(s * RPT, RPT)])


def _make_agg(nch):
  return functools.partial(
      pl.kernel,
      out_type=jax.ShapeDtypeStruct((NC, NPART, D), F32),
      mesh=_mesh,
      scratch_types=[
          pltpu.VMEM((BCH, K), jnp.int32),
          pltpu.VMEM((BCH, K), jnp.int32),
          pltpu.VMEM((NB, KK, D), F32),
          pltpu.VMEM_SHARED((ACCR, D), F32),
      ] + [pltpu.SemaphoreType.DMA] * 16,
  )(functools.partial(_agg_body, nch))


def _deg_body(nch, dst2, ones, zrows, out, didx, ones_v, acc, ssem):
  c = lax.axis_index("c")
  s = lax.axis_index("s")
  wid = s * NC + c

  pltpu.sync_copy(dst2.at[pl.ds(wid * nch, nch)], didx)
  pltpu.sync_copy(ones, ones_v)
  pltpu.sync_copy(zrows, acc.at[pl.ds(s * RPT, RPT)])
  plsc.subcore_barrier()

  def body(u, carry):
    for j in range(4):
      pltpu.async_copy(ones_v, acc.at[didx.at[u * 4 + j]], ssem, add=True)
    for j in range(4):
      pltpu.make_async_copy(ones_v, acc.at[didx.at[u * 4 + j]], ssem).wait()
    return carry

  lax.fori_loop(0, nch // 4, body, 0)
  plsc.subcore_barrier()

  pltpu.sync_copy(acc.at[pl.ds(s * RPT, RPT)], out.at[c, pl.ds(s * RPT, RPT)])


def _make_deg(nch):
  return functools.partial(
      pl.kernel,
      out_type=jax.ShapeDtypeStruct((NC, NPART), jnp.int32),
      mesh=_mesh,
      scratch_types=[
          pltpu.VMEM((nch, K), jnp.int32),
          pltpu.VMEM((K,), jnp.int32),
          pltpu.VMEM_SHARED((ACCR,), jnp.int32),
          pltpu.SemaphoreType.DMA,
      ],
  )(functools.partial(_deg_body, nch))


_RB = 1280
_GRID = NPART // _RB


def _k1_body(degcol, x, w, hs, dis16):
  dis = lax.rsqrt(degcol[:])
  hs[:] = jnp.dot(x[:], w[:], preferred_element_type=F32) * dis
  dis16[:] = jnp.broadcast_to(dis, (_RB, 16))


def _k1_call(degcol, xp, w):
  return pl.pallas_call(
      _k1_body,
      grid=(_GRID,),
      in_specs=[
          pl.BlockSpec((_RB, 1), lambda i: (i, 0)),
          pl.BlockSpec((_RB, D), lambda i: (i, 0)),
          pl.BlockSpec((D, D), lambda i: (0, 0)),
      ],
      out_specs=[
          pl.BlockSpec((_RB, D), lambda i: (i, 0)),
          pl.BlockSpec((_RB, 16), lambda i: (i, 0)),
      ],
      out_shape=[
          jax.ShapeDtypeStruct((NPART, D), F32),
          jax.ShapeDtypeStruct((NPART, 16), F32),
      ],
  )(degcol, xp, w)


def _k2_body(dis16, acc, hsp, b, w, o):
  i = pl.program_id(0)
  dis = dis16[:, 0:1]
  u = jnp.maximum(dis * (acc[0] + acc[1] + hsp[:]) + b[:], 0.0)
  row = i * _RB + lax.broadcasted_iota(jnp.int32, (_RB, D), 0)
  u = jnp.where(row < N, u, 0.0)
  o[:] = jnp.dot(u, w[:], preferred_element_type=F32) * dis


def _k2_call(dis16, acc, hsp, b, w):
  return pl.pallas_call(
      _k2_body,
      grid=(_GRID,),
      in_specs=[
          pl.BlockSpec((_RB, 16), lambda i: (i, 0)),
          pl.BlockSpec((2, _RB, D), lambda i: (0, i, 0)),
          pl.BlockSpec((_RB, D), lambda i: (i, 0)),
          pl.BlockSpec((1, D), lambda i: (0, 0)),
          pl.BlockSpec((D, D), lambda i: (0, 0)),
      ],
      out_specs=pl.BlockSpec((_RB, D), lambda i: (i, 0)),
      out_shape=jax.ShapeDtypeStruct((NPART, D), F32),
  )(dis16, acc, hsp, b, w)


_RB3 = 1000
_GRID3 = N // _RB3


def _k3_body(dis16, acc, hs3, b, o):
  dis = dis16[:, 0:1]
  t = dis * (acc[0] + acc[1] + hs3[:]) + b[:]
  col = lax.broadcasted_iota(jnp.int32, t.shape, 1)
  valid = col < NCLS
  tm = jnp.where(valid, t, -jnp.inf)
  m = jnp.max(tm, axis=1, keepdims=True)
  e = jnp.where(valid, jnp.exp(t - m), 0.0)
  lse = jnp.log(jnp.sum(e, axis=1, keepdims=True))
  o[:] = (t - m - lse)[:, :NCLS]


def _k3_call(dis16, acc, hs3, b):
  return pl.pallas_call(
      _k3_body,
      grid=(_GRID3,),
      in_specs=[
          pl.BlockSpec((_RB3, 16), lambda i: (i, 0)),
          pl.BlockSpec((2, _RB3, D), lambda i: (0, i, 0)),
          pl.BlockSpec((_RB3, D), lambda i: (i, 0)),
          pl.BlockSpec((1, D), lambda i: (0, 0)),
      ],
      out_specs=pl.BlockSpec((_RB3, NCLS), lambda i: (i, 0)),
      out_shape=jax.ShapeDtypeStruct((N, NCLS), F32),
  )(dis16, acc, hs3, b)


def kernel(x, edge_index, W1, b1, W2, b2, W3, b3):
  E = edge_index.shape[1]
  nch = -(-E // (NW * K))
  nch = -(-nch // BCH) * BCH
  e_pad = nch * NW * K

  pad_ar = jnp.arange(e_pad - E, dtype=jnp.int32)
  src2 = jnp.concatenate(
      [edge_index[0], N + pad_ar % (TPAD - N)]).reshape(-1, K)
  dst2 = jnp.concatenate(
      [edge_index[1], N + pad_ar % (NPART - N)]).reshape(-1, K)

  zrows = jnp.zeros((RPT, D), F32)
  zcnt = jnp.zeros((RPT,), jnp.int32)
  ones1 = jnp.ones((K,), jnp.int32)
  xp = jnp.zeros((NPART, D), F32).at[:N].set(x)

  agg = _make_agg(nch)
  deg = _make_deg(nch)

  cnt = deg(dst2, ones1, zcnt)
  degcol = (cnt[0] + cnt[1] + 1).astype(F32).reshape(NPART, 1)
  hs1, dis16 = _k1_call(degcol, xp, W1)
  acc1 = agg(hs1, src2, dst2, zrows)
  hs2 = _k2_call(dis16, acc1, hs1, b1.reshape(1, D), W2)
  acc2 = agg(hs2, src2, dst2, zrows)

  W3p = jnp.zeros((D, D), F32).at[:, :NCLS].set(W3)
  b3p = jnp.zeros((1, D), F32).at[0, :NCLS].set(b3)
  hs3 = _k2_call(dis16, acc2, hs2, b2.reshape(1, D), W3p)
  acc3 = agg(hs3, src2, dst2, zrows)

  return _k3_call(dis16, acc3, hs3, b3p)
